# baseline (speedup 1.0000x reference)
import functools

import jax
import jax.numpy as jnp
from jax import lax
from jax.experimental import pallas as pl
from jax.experimental.pallas import tpu as pltpu
from jax.experimental.pallas import tpu_sc as plsc

_B = 16384
_D = 32
_V = 1000000
_LANES = 16
_NC = 2
_NS = 16
_NW = _NC * _NS
_MW = 244 * 128
_CL = 512
_NT = 62
_TAIL = _NW * _MW + _CL
_NTAIL = _V - _TAIL
_SELCAP = 768
_NSV = _SELCAP // _LANES
_NSLOT = _NW * _SELCAP
_BW = _B // _NW
_BIG = 0x3FFFFFFF


def _mesh():
    return plsc.VectorSubcoreMesh(
        core_axis_name="c", subcore_axis_name="s",
        num_cores=_NC, num_subcores=_NS,
    )


def _build_k1():
    @functools.partial(
        pl.kernel,
        out_type=(jax.ShapeDtypeStruct((_NSLOT * _D,), jnp.float32),
                  jax.ShapeDtypeStruct((_NSLOT * _D,), jnp.float32),
                  jax.ShapeDtypeStruct((_NSLOT,), jnp.int32),
                  jax.ShapeDtypeStruct((_NSLOT,), jnp.int32)),
        mesh=_mesh(),
        compiler_params=pltpu.CompilerParams(needs_layout_passes=False),
        scratch_types=[
            pltpu.VMEM((2, _D, _CL), jnp.float32),
            pltpu.VMEM((2, _D, _CL), jnp.float32),
            pltpu.VMEM((_SELCAP * _D,), jnp.float32),
            pltpu.VMEM((_SELCAP * _D,), jnp.float32),
            pltpu.VMEM((_SELCAP,), jnp.int32),
            pltpu.VMEM((_SELCAP,), jnp.int32),
            pltpu.VMEM((_SELCAP,), jnp.int32),
            pltpu.VMEM((_SELCAP,), jnp.int32),
            pltpu.VMEM((_CL,), jnp.int32),
            pltpu.VMEM((_CL,), jnp.int32),
            pltpu.VMEM((48,), jnp.int32),
            pltpu.VMEM((48,), jnp.int32),
            pltpu.SemaphoreType.DMA,
            pltpu.SemaphoreType.DMA,
            pltpu.SemaphoreType.DMA,
            pltpu.SemaphoreType.DMA,
        ],
    )
    def k1(user_ids, item_ids, u2, i2, ug, ig, bmap_u, bmap_i,
           ubuf, ibuf, rows_u, rows_i, pk_u, pk_i, b_u, b_i,
           idb_u, idb_i, wcol, wslot, su0, su1, si0, si1):
        wid = lax.axis_index("s") * _NC + lax.axis_index("c")
        base = wid * _MW
        hi = jnp.where(wid == _NW - 1, _TAIL, base + _MW)
        lane = lax.iota(jnp.int32, _LANES)
        su = (su0, su1)
        si = (si0, si1)

        for k in range(_NSV):
            sl = pl.ds(k * _LANES, _LANES)
            pk_u[sl] = jnp.full((_LANES,), _BIG, jnp.int32)
            pk_i[sl] = jnp.full((_LANES,), _BIG, jnp.int32)
            pad = jnp.full((_LANES,), _B, jnp.int32)
            b_u[sl] = pad
            b_i[sl] = pad

        for t in (0, 1):
            sl = pl.ds(pl.multiple_of(base + t * _CL, 128), _CL)
            pltpu.async_copy(u2.at[:, sl], ubuf.at[t], su[t])
            pltpu.async_copy(i2.at[:, sl], ibuf.at[t], si[t])

        def sel_body(p, offs):
            off_u, off_i = offs
            pltpu.sync_copy(user_ids.at[pl.ds(p * _CL, _CL)], idb_u)
            pltpu.sync_copy(item_ids.at[pl.ds(p * _CL, _CL)], idb_i)
            for v in range(_CL // _LANES):
                bvec = p * _CL + v * _LANES + lane
                idv = idb_u[pl.ds(v * _LANES, _LANES)]
                m = (idv >= base) & (idv < hi)
                plsc.store_compressed(pk_u.at[pl.ds(off_u, _LANES)],
                                      idv - base, mask=m)
                plsc.store_compressed(b_u.at[pl.ds(off_u, _LANES)],
                                      bvec, mask=m)
                off_u = off_u + jnp.sum(m.astype(jnp.int32))
                idv = idb_i[pl.ds(v * _LANES, _LANES)]
                m = (idv >= base) & (idv < hi)
                plsc.store_compressed(pk_i.at[pl.ds(off_i, _LANES)],
                                      idv - base, mask=m)
                plsc.store_compressed(b_i.at[pl.ds(off_i, _LANES)],
                                      bvec, mask=m)
                off_i = off_i + jnp.sum(m.astype(jnp.int32))
            return off_u, off_i

        lax.fori_loop(0, _B // _CL, sel_body, (jnp.int32(0), jnp.int32(0)))

        def extract(buf, bank, pk, rows, t):
            nw = jnp.int32(0)
            for v in range(_NSV):
                pk16 = pk[pl.ds(v * _LANES, _LANES)]
                m = (pk16 >> 9) == t
                plsc.store_compressed(wcol.at[pl.ds(nw, _LANES)],
                                      pk16 & 511, mask=m)
                plsc.store_compressed(wslot.at[pl.ds(nw, _LANES)],
                                      v * _LANES + lane, mask=m)
                nw = nw + jnp.sum(m.astype(jnp.int32))
            bank16 = jnp.full((_LANES,), bank, jnp.int32)
            for k in range(3):
                valid = (k * _LANES + lane) < nw
                col = wcol[pl.ds(k * _LANES, _LANES)]
                sbase = wslot[pl.ds(k * _LANES, _LANES)] * _D
                for d in range(_D):
                    d16 = jnp.full((_LANES,), d, jnp.int32)
                    val = plsc.load_gather(buf, [bank16, d16, col],
                                           mask=valid)
                    plsc.store_scatter(rows, [sbase + d], val, mask=valid)

        basea = pl.multiple_of(base, 128)

        def stream_body(g, carry):
            for parity in (0, 1):
                t = 2 * g + parity
                nxt = pl.multiple_of(
                    base + jnp.minimum(t + 2, _NT - 1) * _CL, 128)
                pltpu.make_async_copy(
                    u2.at[:, pl.ds(basea, _CL)], ubuf.at[parity],
                    su[parity]).wait()
                extract(ubuf, parity, pk_u, rows_u, t)
                pltpu.async_copy(u2.at[:, pl.ds(nxt, _CL)],
                                 ubuf.at[parity], su[parity])
                pltpu.make_async_copy(
                    i2.at[:, pl.ds(basea, _CL)], ibuf.at[parity],
                    si[parity]).wait()
                extract(ibuf, parity, pk_i, rows_i, t)
                pltpu.async_copy(i2.at[:, pl.ds(nxt, _CL)],
                                 ibuf.at[parity], si[parity])
            return carry

        lax.fori_loop(0, _NT // 2, stream_body, jnp.int32(0))
        for parity in (0, 1):
            pltpu.make_async_copy(u2.at[:, pl.ds(basea, _CL)],
                                  ubuf.at[parity], su[parity]).wait()
            pltpu.make_async_copy(i2.at[:, pl.ds(basea, _CL)],
                                  ibuf.at[parity], si[parity]).wait()

        pltpu.sync_copy(rows_u, ug.at[pl.ds(wid * _SELCAP * _D,
                                            _SELCAP * _D)])
        pltpu.sync_copy(rows_i, ig.at[pl.ds(wid * _SELCAP * _D,
                                            _SELCAP * _D)])
        pltpu.sync_copy(b_u, bmap_u.at[pl.ds(wid * _SELCAP, _SELCAP)])
        pltpu.sync_copy(b_i, bmap_i.at[pl.ds(wid * _SELCAP, _SELCAP)])

    return k1


def _build_k2():
    @functools.partial(
        pl.kernel,
        out_type=jax.ShapeDtypeStruct((_B,), jnp.float32),
        mesh=_mesh(),
        compiler_params=pltpu.CompilerParams(
            needs_layout_passes=False, use_tc_tiling_on_sc=False),
        scratch_types=[
            pltpu.VMEM((4, 128), jnp.int32),
            pltpu.VMEM((4, 128), jnp.int32),
            pltpu.VMEM((_BW,), jnp.int32),
            pltpu.VMEM((_BW,), jnp.int32),
            pltpu.VMEM((_NSLOT,), jnp.int32),
            pltpu.VMEM((4, 128), jnp.int32),
            pltpu.VMEM((4, 128), jnp.int32),
            pltpu.VMEM((_BW, _D), jnp.float32),
            pltpu.VMEM((_BW, _D), jnp.float32),
            pltpu.VMEM((_BW,), jnp.float32),
            pltpu.VMEM((_BW,), jnp.float32),
            pltpu.VMEM((_NTAIL, _D), jnp.float32),
            pltpu.VMEM((_NTAIL, _D), jnp.float32),
            pltpu.VMEM((_BW,), jnp.float32),
            pltpu.SemaphoreType.DMA,
        ],
    )
    def k2(user_ids, item_ids, ug, ig, bmap_u, bmap_i, u_tail, i_tail,
           user_bias, item_bias, out_hbm,
           uid_v, iid_v, uidf, iidf, bm_v, slot_u, slot_i, urows, irows,
           ubf, ibf, ut_v, it_v, out_v, sem):
        wid = lax.axis_index("s") * _NC + lax.axis_index("c")
        base = wid * _BW
        lane = lax.iota(jnp.int32, _LANES)
        for j in range(4):
            pltpu.sync_copy(user_ids.at[pl.ds(base + j * 128, 128)],
                            uid_v.at[j])
            pltpu.sync_copy(item_ids.at[pl.ds(base + j * 128, 128)],
                            iid_v.at[j])
        pltpu.sync_copy(user_ids.at[pl.ds(base, _BW)], uidf)
        pltpu.sync_copy(item_ids.at[pl.ds(base, _BW)], iidf)
        pltpu.sync_copy(u_tail, ut_v)
        pltpu.sync_copy(i_tail, it_v)

        ubsrc = user_bias.at[0]
        ibsrc = item_bias.at[0]
        cps = []
        for j in range(4):
            sl = pl.ds(j * 128, 128)
            cps.append(pltpu.async_copy(
                ubsrc.at[uid_v.at[j]], ubf.at[sl], sem))
            cps.append(pltpu.async_copy(
                ibsrc.at[iid_v.at[j]], ibf.at[sl], sem))
        for c in cps:
            c.wait()

        def invert(bmap, slot2d):
            for j in range(4):
                j16 = jnp.full((_LANES,), j, jnp.int32)
                for k in range(8):
                    plsc.store_scatter(slot2d, [j16, k * _LANES + lane],
                                       jnp.zeros((_LANES,), jnp.int32))
            pltpu.sync_copy(bmap, bm_v)

            def inv_body(v, carry):
                bm16 = bm_v[pl.ds(v * _LANES, _LANES)]
                m = (bm16 >= base) & (bm16 < base + _BW)
                r = bm16 - base
                plsc.store_scatter(slot2d, [r >> 7, r & 127],
                                   v * _LANES + lane, mask=m)
                return carry

            lax.fori_loop(0, _NSLOT // _LANES, inv_body, jnp.int32(0))

        invert(bmap_u, slot_u)
        invert(bmap_i, slot_i)

        cps = []
        for j in range(4):
            sl = pl.ds(j * 128, 128)
            cps.append(pltpu.async_copy(
                ug.at[slot_u.at[j]], urows.at[sl], sem))
            cps.append(pltpu.async_copy(
                ig.at[slot_i.at[j]], irows.at[sl], sem))
        for c in cps:
            c.wait()

        def group(g, carry):
            row = g * _LANES + lane
            sl16 = pl.ds(g * _LANES, _LANES)
            acc = ubf[sl16] + ibf[sl16]
            for d in range(_D):
                col = jnp.full((_LANES,), d, jnp.int32)
                acc = acc + (plsc.load_gather(urows, [row, col])
                             * plsc.load_gather(irows, [row, col]))
            out_v[sl16] = acc
            idu = uidf[sl16]
            idi = iidf[sl16]
            mu = idu >= _TAIL
            mi = idi >= _TAIL
            anyt = jnp.sum((mu | mi).astype(jnp.int32))

            @pl.when(anyt > 0)
            def _():
                tru = idu - _TAIL
                tri = idi - _TAIL
                acc2 = ubf[sl16] + ibf[sl16]
                for d in range(_D):
                    col = jnp.full((_LANES,), d, jnp.int32)
                    uv = plsc.load_gather(urows, [row, col])
                    utv = plsc.load_gather(ut_v, [tru, col], mask=mu)
                    uv = jnp.where(mu, utv, uv)
                    iv = plsc.load_gather(irows, [row, col])
                    itv = plsc.load_gather(it_v, [tri, col], mask=mi)
                    iv = jnp.where(mi, itv, iv)
                    acc2 = acc2 + uv * iv
                out_v[sl16] = jnp.where(mu | mi, acc2, out_v[sl16])

            return carry

        lax.fori_loop(0, _BW // _LANES, group, jnp.int32(0))
        pltpu.sync_copy(out_v, out_hbm.at[pl.ds(base, _BW)])

    return k2


def kernel(user_ids, item_ids, user_emb, item_emb, user_bias, item_bias):
    ug1, ig1, bmap_u, bmap_i = _build_k1()(
        user_ids, item_ids, user_emb.T, item_emb.T)
    tail_idx = jnp.arange(_TAIL, _V, dtype=jnp.int32)
    u_tail = jnp.take(user_emb, tail_idx, axis=0)
    i_tail = jnp.take(item_emb, tail_idx, axis=0)
    return _build_k2()(
        user_ids, item_ids,
        ug1.reshape(_NSLOT, _D), ig1.reshape(_NSLOT, _D),
        bmap_u, bmap_i, u_tail, i_tail,
        user_bias.T, item_bias.T)


# speedup vs baseline: 1.0594x; 1.0594x over previous
import functools

import jax
import jax.numpy as jnp
from jax import lax
from jax.experimental import pallas as pl
from jax.experimental.pallas import tpu as pltpu
from jax.experimental.pallas import tpu_sc as plsc

_B = 16384
_D = 32
_V = 1000000
_LANES = 16
_NC = 2
_NS = 16
_NW = _NC * _NS
_MW = 244 * 128
_CL = 512
_NT = 62
_TAIL = _NW * _MW + _CL
_NTAIL = _V - _TAIL
_SELCAP = 768
_NSV = _SELCAP // _LANES
_NSLOT = _NW * _SELCAP
_BW = _B // _NW
_BIG = 0x3FFFFFFF


def _mesh():
    return plsc.VectorSubcoreMesh(
        core_axis_name="c", subcore_axis_name="s",
        num_cores=_NC, num_subcores=_NS,
    )


def _build_k1():
    @functools.partial(
        pl.kernel,
        out_type=(jax.ShapeDtypeStruct((_NSLOT * _D,), jnp.float32),
                  jax.ShapeDtypeStruct((_NSLOT * _D,), jnp.float32),
                  jax.ShapeDtypeStruct((_NSLOT,), jnp.int32),
                  jax.ShapeDtypeStruct((_NSLOT,), jnp.int32)),
        mesh=_mesh(),
        compiler_params=pltpu.CompilerParams(needs_layout_passes=False),
        scratch_types=[
            pltpu.VMEM((2, _D, _CL), jnp.float32),
            pltpu.VMEM((2, _D, _CL), jnp.float32),
            pltpu.VMEM((_SELCAP * _D,), jnp.float32),
            pltpu.VMEM((_SELCAP * _D,), jnp.float32),
            pltpu.VMEM((_SELCAP,), jnp.int32),
            pltpu.VMEM((_SELCAP,), jnp.int32),
            pltpu.VMEM((_SELCAP,), jnp.int32),
            pltpu.VMEM((_SELCAP,), jnp.int32),
            pltpu.VMEM((_CL,), jnp.int32),
            pltpu.VMEM((_CL,), jnp.int32),
            pltpu.VMEM((48,), jnp.int32),
            pltpu.VMEM((48,), jnp.int32),
            pltpu.SemaphoreType.DMA,
            pltpu.SemaphoreType.DMA,
            pltpu.SemaphoreType.DMA,
            pltpu.SemaphoreType.DMA,
        ],
    )
    def k1(user_ids, item_ids, u2, i2, ug, ig, bmap_u, bmap_i,
           ubuf, ibuf, rows_u, rows_i, pk_u, pk_i, b_u, b_i,
           idb_u, idb_i, wcol, wslot, su0, su1, si0, si1):
        wid = lax.axis_index("s") * _NC + lax.axis_index("c")
        base = wid * _MW
        hi = jnp.where(wid == _NW - 1, _TAIL, base + _MW)
        lane = lax.iota(jnp.int32, _LANES)
        su = (su0, su1)
        si = (si0, si1)

        for k in range(_NSV):
            sl = pl.ds(k * _LANES, _LANES)
            pk_u[sl] = jnp.full((_LANES,), _BIG, jnp.int32)
            pk_i[sl] = jnp.full((_LANES,), _BIG, jnp.int32)
            pad = jnp.full((_LANES,), _B, jnp.int32)
            b_u[sl] = pad
            b_i[sl] = pad

        for t in (0, 1):
            sl = pl.ds(pl.multiple_of(base + t * _CL, 128), _CL)
            pltpu.async_copy(u2.at[:, sl], ubuf.at[t], su[t])
            pltpu.async_copy(i2.at[:, sl], ibuf.at[t], si[t])

        def sel_body(p, offs):
            off_u, off_i = offs
            pltpu.sync_copy(user_ids.at[pl.ds(p * _CL, _CL)], idb_u)
            pltpu.sync_copy(item_ids.at[pl.ds(p * _CL, _CL)], idb_i)
            for v in range(_CL // _LANES):
                bvec = p * _CL + v * _LANES + lane
                idv = idb_u[pl.ds(v * _LANES, _LANES)]
                m = (idv >= base) & (idv < hi)
                plsc.store_compressed(pk_u.at[pl.ds(off_u, _LANES)],
                                      idv - base, mask=m)
                plsc.store_compressed(b_u.at[pl.ds(off_u, _LANES)],
                                      bvec, mask=m)
                off_u = off_u + jnp.sum(m.astype(jnp.int32))
                idv = idb_i[pl.ds(v * _LANES, _LANES)]
                m = (idv >= base) & (idv < hi)
                plsc.store_compressed(pk_i.at[pl.ds(off_i, _LANES)],
                                      idv - base, mask=m)
                plsc.store_compressed(b_i.at[pl.ds(off_i, _LANES)],
                                      bvec, mask=m)
                off_i = off_i + jnp.sum(m.astype(jnp.int32))
            return off_u, off_i

        lax.fori_loop(0, _B // _CL, sel_body, (jnp.int32(0), jnp.int32(0)))

        def extract(buf, bank, pk, rows, t):
            nw = jnp.int32(0)
            for v in range(_NSV):
                pk16 = pk[pl.ds(v * _LANES, _LANES)]
                m = (pk16 >> 9) == t
                plsc.store_compressed(wcol.at[pl.ds(nw, _LANES)],
                                      pk16 & 511, mask=m)
                plsc.store_compressed(wslot.at[pl.ds(nw, _LANES)],
                                      v * _LANES + lane, mask=m)
                nw = nw + jnp.sum(m.astype(jnp.int32))
            bank16 = jnp.full((_LANES,), bank, jnp.int32)
            for k in range(3):
                def grp(k=k):
                    valid = (k * _LANES + lane) < nw
                    col = wcol[pl.ds(k * _LANES, _LANES)]
                    sbase = wslot[pl.ds(k * _LANES, _LANES)] * _D
                    for d in range(_D):
                        d16 = jnp.full((_LANES,), d, jnp.int32)
                        val = plsc.load_gather(buf, [bank16, d16, col],
                                               mask=valid)
                        plsc.store_scatter(rows, [sbase + d], val,
                                           mask=valid)
                if k == 0:
                    grp()
                else:
                    pl.when(nw > k * _LANES)(grp)

        basea = pl.multiple_of(base, 128)

        def stream_body(g, carry):
            for parity in (0, 1):
                t = 2 * g + parity
                nxt = pl.multiple_of(
                    base + jnp.minimum(t + 2, _NT - 1) * _CL, 128)
                pltpu.make_async_copy(
                    u2.at[:, pl.ds(basea, _CL)], ubuf.at[parity],
                    su[parity]).wait()
                extract(ubuf, parity, pk_u, rows_u, t)
                pltpu.async_copy(u2.at[:, pl.ds(nxt, _CL)],
                                 ubuf.at[parity], su[parity])
                pltpu.make_async_copy(
                    i2.at[:, pl.ds(basea, _CL)], ibuf.at[parity],
                    si[parity]).wait()
                extract(ibuf, parity, pk_i, rows_i, t)
                pltpu.async_copy(i2.at[:, pl.ds(nxt, _CL)],
                                 ibuf.at[parity], si[parity])
            return carry

        lax.fori_loop(0, _NT // 2, stream_body, jnp.int32(0))
        for parity in (0, 1):
            pltpu.make_async_copy(u2.at[:, pl.ds(basea, _CL)],
                                  ubuf.at[parity], su[parity]).wait()
            pltpu.make_async_copy(i2.at[:, pl.ds(basea, _CL)],
                                  ibuf.at[parity], si[parity]).wait()

        pltpu.sync_copy(rows_u, ug.at[pl.ds(wid * _SELCAP * _D,
                                            _SELCAP * _D)])
        pltpu.sync_copy(rows_i, ig.at[pl.ds(wid * _SELCAP * _D,
                                            _SELCAP * _D)])
        pltpu.sync_copy(b_u, bmap_u.at[pl.ds(wid * _SELCAP, _SELCAP)])
        pltpu.sync_copy(b_i, bmap_i.at[pl.ds(wid * _SELCAP, _SELCAP)])

    return k1


def _build_k2():
    @functools.partial(
        pl.kernel,
        out_type=jax.ShapeDtypeStruct((_B,), jnp.float32),
        mesh=_mesh(),
        compiler_params=pltpu.CompilerParams(
            needs_layout_passes=False, use_tc_tiling_on_sc=False),
        scratch_types=[
            pltpu.VMEM((4, 128), jnp.int32),
            pltpu.VMEM((4, 128), jnp.int32),
            pltpu.VMEM((_BW,), jnp.int32),
            pltpu.VMEM((_BW,), jnp.int32),
            pltpu.VMEM((_NSLOT,), jnp.int32),
            pltpu.VMEM((4, 128), jnp.int32),
            pltpu.VMEM((4, 128), jnp.int32),
            pltpu.VMEM((_BW, _D), jnp.float32),
            pltpu.VMEM((_BW, _D), jnp.float32),
            pltpu.VMEM((_BW,), jnp.float32),
            pltpu.VMEM((_BW,), jnp.float32),
            pltpu.VMEM((_NTAIL, _D), jnp.float32),
            pltpu.VMEM((_NTAIL, _D), jnp.float32),
            pltpu.VMEM((_BW,), jnp.float32),
            pltpu.SemaphoreType.DMA,
        ],
    )
    def k2(user_ids, item_ids, ug, ig, bmap_u, bmap_i, u_tail, i_tail,
           user_bias, item_bias, out_hbm,
           uid_v, iid_v, uidf, iidf, bm_v, slot_u, slot_i, urows, irows,
           ubf, ibf, ut_v, it_v, out_v, sem):
        wid = lax.axis_index("s") * _NC + lax.axis_index("c")
        base = wid * _BW
        lane = lax.iota(jnp.int32, _LANES)
        for j in range(4):
            pltpu.sync_copy(user_ids.at[pl.ds(base + j * 128, 128)],
                            uid_v.at[j])
            pltpu.sync_copy(item_ids.at[pl.ds(base + j * 128, 128)],
                            iid_v.at[j])
        pltpu.sync_copy(user_ids.at[pl.ds(base, _BW)], uidf)
        pltpu.sync_copy(item_ids.at[pl.ds(base, _BW)], iidf)
        pltpu.sync_copy(u_tail, ut_v)
        pltpu.sync_copy(i_tail, it_v)

        ubsrc = user_bias.at[0]
        ibsrc = item_bias.at[0]
        cps = []
        for j in range(4):
            sl = pl.ds(j * 128, 128)
            cps.append(pltpu.async_copy(
                ubsrc.at[uid_v.at[j]], ubf.at[sl], sem))
            cps.append(pltpu.async_copy(
                ibsrc.at[iid_v.at[j]], ibf.at[sl], sem))
        for c in cps:
            c.wait()

        def invert(bmap, slot2d):
            for j in range(4):
                j16 = jnp.full((_LANES,), j, jnp.int32)
                for k in range(8):
                    plsc.store_scatter(slot2d, [j16, k * _LANES + lane],
                                       jnp.zeros((_LANES,), jnp.int32))
            pltpu.sync_copy(bmap, bm_v)

            def inv_body(v, carry):
                bm16 = bm_v[pl.ds(v * _LANES, _LANES)]
                m = (bm16 >= base) & (bm16 < base + _BW)
                r = bm16 - base
                plsc.store_scatter(slot2d, [r >> 7, r & 127],
                                   v * _LANES + lane, mask=m)
                return carry

            lax.fori_loop(0, _NSLOT // _LANES, inv_body, jnp.int32(0))

        invert(bmap_u, slot_u)
        invert(bmap_i, slot_i)

        cps = []
        for j in range(4):
            sl = pl.ds(j * 128, 128)
            cps.append(pltpu.async_copy(
                ug.at[slot_u.at[j]], urows.at[sl], sem))
            cps.append(pltpu.async_copy(
                ig.at[slot_i.at[j]], irows.at[sl], sem))
        for c in cps:
            c.wait()

        def group(g, carry):
            row = g * _LANES + lane
            sl16 = pl.ds(g * _LANES, _LANES)
            acc = ubf[sl16] + ibf[sl16]
            for d in range(_D):
                col = jnp.full((_LANES,), d, jnp.int32)
                acc = acc + (plsc.load_gather(urows, [row, col])
                             * plsc.load_gather(irows, [row, col]))
            out_v[sl16] = acc
            idu = uidf[sl16]
            idi = iidf[sl16]
            mu = idu >= _TAIL
            mi = idi >= _TAIL
            anyt = jnp.sum((mu | mi).astype(jnp.int32))

            @pl.when(anyt > 0)
            def _():
                tru = idu - _TAIL
                tri = idi - _TAIL
                acc2 = ubf[sl16] + ibf[sl16]
                for d in range(_D):
                    col = jnp.full((_LANES,), d, jnp.int32)
                    uv = plsc.load_gather(urows, [row, col])
                    utv = plsc.load_gather(ut_v, [tru, col], mask=mu)
                    uv = jnp.where(mu, utv, uv)
                    iv = plsc.load_gather(irows, [row, col])
                    itv = plsc.load_gather(it_v, [tri, col], mask=mi)
                    iv = jnp.where(mi, itv, iv)
                    acc2 = acc2 + uv * iv
                out_v[sl16] = jnp.where(mu | mi, acc2, out_v[sl16])

            return carry

        lax.fori_loop(0, _BW // _LANES, group, jnp.int32(0))
        pltpu.sync_copy(out_v, out_hbm.at[pl.ds(base, _BW)])

    return k2


def kernel(user_ids, item_ids, user_emb, item_emb, user_bias, item_bias):
    ug1, ig1, bmap_u, bmap_i = _build_k1()(
        user_ids, item_ids, user_emb.T, item_emb.T)
    tail_idx = jnp.arange(_TAIL, _V, dtype=jnp.int32)
    u_tail = jnp.take(user_emb, tail_idx, axis=0)
    i_tail = jnp.take(item_emb, tail_idx, axis=0)
    return _build_k2()(
        user_ids, item_ids,
        ug1.reshape(_NSLOT, _D), ig1.reshape(_NSLOT, _D),
        bmap_u, bmap_i, u_tail, i_tail,
        user_bias.T, item_bias.T)


# speedup vs baseline: 1.0669x; 1.0071x over previous
import functools

import jax
import jax.numpy as jnp
from jax import lax
from jax.experimental import pallas as pl
from jax.experimental.pallas import tpu as pltpu
from jax.experimental.pallas import tpu_sc as plsc

_B = 16384
_D = 32
_V = 1000000
_LANES = 16
_NC = 2
_NS = 16
_NW = _NC * _NS
_MW = 244 * 128
_CL = 512
_NT = 62
_TAIL = _NW * _MW + _CL
_NTAIL = _V - _TAIL
_SELCAP = 768
_NSV = _SELCAP // _LANES
_NSLOT = _NW * _SELCAP
_BW = _B // _NW
_BIG = 0x3FFFFFFF


def _mesh():
    return plsc.VectorSubcoreMesh(
        core_axis_name="c", subcore_axis_name="s",
        num_cores=_NC, num_subcores=_NS,
    )


def _build_k1():
    @functools.partial(
        pl.kernel,
        out_type=(jax.ShapeDtypeStruct((_NSLOT * _D,), jnp.float32),
                  jax.ShapeDtypeStruct((_NSLOT * _D,), jnp.float32),
                  jax.ShapeDtypeStruct((_NSLOT,), jnp.int32),
                  jax.ShapeDtypeStruct((_NSLOT,), jnp.int32)),
        mesh=_mesh(),
        compiler_params=pltpu.CompilerParams(needs_layout_passes=False),
        scratch_types=[
            pltpu.VMEM((2, _D, _CL), jnp.float32),
            pltpu.VMEM((2, _D, _CL), jnp.float32),
            pltpu.VMEM((_SELCAP * _D,), jnp.float32),
            pltpu.VMEM((_SELCAP * _D,), jnp.float32),
            pltpu.VMEM((_SELCAP,), jnp.int32),
            pltpu.VMEM((_SELCAP,), jnp.int32),
            pltpu.VMEM((_SELCAP,), jnp.int32),
            pltpu.VMEM((_SELCAP,), jnp.int32),
            pltpu.VMEM((_CL,), jnp.int32),
            pltpu.VMEM((_CL,), jnp.int32),
            pltpu.VMEM((48,), jnp.int32),
            pltpu.VMEM((48,), jnp.int32),
            pltpu.SemaphoreType.DMA,
            pltpu.SemaphoreType.DMA,
            pltpu.SemaphoreType.DMA,
            pltpu.SemaphoreType.DMA,
        ],
    )
    def k1(user_ids, item_ids, u2, i2, ug, ig, bmap_u, bmap_i,
           ubuf, ibuf, rows_u, rows_i, pk_u, pk_i, b_u, b_i,
           idb_u, idb_i, wcol, wslot, su0, su1, si0, si1):
        wid = lax.axis_index("s") * _NC + lax.axis_index("c")
        base = wid * _MW
        hi = jnp.where(wid == _NW - 1, _TAIL, base + _MW)
        lane = lax.iota(jnp.int32, _LANES)
        su = (su0, su1)
        si = (si0, si1)

        for k in range(_NSV):
            sl = pl.ds(k * _LANES, _LANES)
            pk_u[sl] = jnp.full((_LANES,), _BIG, jnp.int32)
            pk_i[sl] = jnp.full((_LANES,), _BIG, jnp.int32)
            pad = jnp.full((_LANES,), _B, jnp.int32)
            b_u[sl] = pad
            b_i[sl] = pad

        for t in (0, 1):
            sl = pl.ds(pl.multiple_of(base + t * _CL, 128), _CL)
            pltpu.async_copy(u2.at[:, sl], ubuf.at[t], su[t])
            pltpu.async_copy(i2.at[:, sl], ibuf.at[t], si[t])

        def sel_body(p, offs):
            off_u, off_i = offs
            pltpu.sync_copy(user_ids.at[pl.ds(p * _CL, _CL)], idb_u)
            pltpu.sync_copy(item_ids.at[pl.ds(p * _CL, _CL)], idb_i)
            for v in range(_CL // _LANES):
                bvec = p * _CL + v * _LANES + lane
                idv = idb_u[pl.ds(v * _LANES, _LANES)]
                m = (idv >= base) & (idv < hi)
                plsc.store_compressed(pk_u.at[pl.ds(off_u, _LANES)],
                                      idv - base, mask=m)
                plsc.store_compressed(b_u.at[pl.ds(off_u, _LANES)],
                                      bvec, mask=m)
                off_u = off_u + jnp.sum(m.astype(jnp.int32))
                idv = idb_i[pl.ds(v * _LANES, _LANES)]
                m = (idv >= base) & (idv < hi)
                plsc.store_compressed(pk_i.at[pl.ds(off_i, _LANES)],
                                      idv - base, mask=m)
                plsc.store_compressed(b_i.at[pl.ds(off_i, _LANES)],
                                      bvec, mask=m)
                off_i = off_i + jnp.sum(m.astype(jnp.int32))
            return off_u, off_i

        lax.fori_loop(0, _B // _CL, sel_body, (jnp.int32(0), jnp.int32(0)))

        def extract(buf, bank, pk, rows, t):
            nw = jnp.int32(0)
            for v in range(_NSV):
                pk16 = pk[pl.ds(v * _LANES, _LANES)]
                m = (pk16 >> 9) == t
                plsc.store_compressed(wcol.at[pl.ds(nw, _LANES)],
                                      pk16 & 511, mask=m)
                plsc.store_compressed(wslot.at[pl.ds(nw, _LANES)],
                                      v * _LANES + lane, mask=m)
                nw = nw + jnp.sum(m.astype(jnp.int32))
            bank16 = jnp.full((_LANES,), bank, jnp.int32)
            for k in range(3):
                def grp(k=k):
                    valid = (k * _LANES + lane) < nw
                    col = wcol[pl.ds(k * _LANES, _LANES)]
                    sbase = wslot[pl.ds(k * _LANES, _LANES)] * _D
                    for d in range(_D):
                        d16 = jnp.full((_LANES,), d, jnp.int32)
                        val = plsc.load_gather(buf, [bank16, d16, col],
                                               mask=valid)
                        plsc.store_scatter(rows, [sbase + d], val,
                                           mask=valid)
                if k == 0:
                    grp()
                else:
                    pl.when(nw > k * _LANES)(grp)

        basea = pl.multiple_of(base, 128)

        def stream_body(g, carry):
            for parity in (0, 1):
                t = 2 * g + parity
                nxt = pl.multiple_of(
                    base + jnp.minimum(t + 2, _NT - 1) * _CL, 128)
                pltpu.make_async_copy(
                    u2.at[:, pl.ds(basea, _CL)], ubuf.at[parity],
                    su[parity]).wait()
                extract(ubuf, parity, pk_u, rows_u, t)
                pltpu.async_copy(u2.at[:, pl.ds(nxt, _CL)],
                                 ubuf.at[parity], su[parity])
                pltpu.make_async_copy(
                    i2.at[:, pl.ds(basea, _CL)], ibuf.at[parity],
                    si[parity]).wait()
                extract(ibuf, parity, pk_i, rows_i, t)
                pltpu.async_copy(i2.at[:, pl.ds(nxt, _CL)],
                                 ibuf.at[parity], si[parity])
            return carry

        lax.fori_loop(0, _NT // 2, stream_body, jnp.int32(0))
        for parity in (0, 1):
            pltpu.make_async_copy(u2.at[:, pl.ds(basea, _CL)],
                                  ubuf.at[parity], su[parity]).wait()
            pltpu.make_async_copy(i2.at[:, pl.ds(basea, _CL)],
                                  ibuf.at[parity], si[parity]).wait()

        pltpu.sync_copy(rows_u, ug.at[pl.ds(wid * _SELCAP * _D,
                                            _SELCAP * _D)])
        pltpu.sync_copy(rows_i, ig.at[pl.ds(wid * _SELCAP * _D,
                                            _SELCAP * _D)])
        pltpu.sync_copy(b_u, bmap_u.at[pl.ds(wid * _SELCAP, _SELCAP)])
        pltpu.sync_copy(b_i, bmap_i.at[pl.ds(wid * _SELCAP, _SELCAP)])

    return k1


def _build_k2():
    @functools.partial(
        pl.kernel,
        out_type=jax.ShapeDtypeStruct((_B,), jnp.float32),
        mesh=_mesh(),
        compiler_params=pltpu.CompilerParams(
            needs_layout_passes=False, use_tc_tiling_on_sc=False),
        scratch_types=[
            pltpu.VMEM((4, 128), jnp.int32),
            pltpu.VMEM((4, 128), jnp.int32),
            pltpu.VMEM((_BW,), jnp.int32),
            pltpu.VMEM((_BW,), jnp.int32),
            pltpu.VMEM((_NSLOT,), jnp.int32),
            pltpu.VMEM((4, 128), jnp.int32),
            pltpu.VMEM((4, 128), jnp.int32),
            pltpu.VMEM((_BW, _D), jnp.float32),
            pltpu.VMEM((_BW, _D), jnp.float32),
            pltpu.VMEM((_BW,), jnp.float32),
            pltpu.VMEM((_BW,), jnp.float32),
            pltpu.VMEM((_NTAIL, _D), jnp.float32),
            pltpu.VMEM((_NTAIL, _D), jnp.float32),
            pltpu.VMEM((_BW,), jnp.float32),
            pltpu.SemaphoreType.DMA,
        ],
    )
    def k2(user_ids, item_ids, ug, ig, bmap_u, bmap_i, u_tail, i_tail,
           user_bias, item_bias, out_hbm,
           uid_v, iid_v, uidf, iidf, bm_v, slot_u, slot_i, urows, irows,
           ubf, ibf, ut_v, it_v, out_v, sem):
        wid = lax.axis_index("s") * _NC + lax.axis_index("c")
        base = wid * _BW
        lane = lax.iota(jnp.int32, _LANES)
        for j in range(4):
            pltpu.sync_copy(user_ids.at[pl.ds(base + j * 128, 128)],
                            uid_v.at[j])
            pltpu.sync_copy(item_ids.at[pl.ds(base + j * 128, 128)],
                            iid_v.at[j])
        pltpu.sync_copy(user_ids.at[pl.ds(base, _BW)], uidf)
        pltpu.sync_copy(item_ids.at[pl.ds(base, _BW)], iidf)
        pltpu.sync_copy(u_tail, ut_v)
        pltpu.sync_copy(i_tail, it_v)

        ubsrc = user_bias.at[0]
        ibsrc = item_bias.at[0]
        cps = []
        for j in range(4):
            sl = pl.ds(j * 128, 128)
            cps.append(pltpu.async_copy(
                ubsrc.at[uid_v.at[j]], ubf.at[sl], sem))
            cps.append(pltpu.async_copy(
                ibsrc.at[iid_v.at[j]], ibf.at[sl], sem))
        for c in cps:
            c.wait()

        def invert(bmap, slot2d):
            for j in range(4):
                j16 = jnp.full((_LANES,), j, jnp.int32)
                for k in range(8):
                    plsc.store_scatter(slot2d, [j16, k * _LANES + lane],
                                       jnp.zeros((_LANES,), jnp.int32))
            pltpu.sync_copy(bmap, bm_v)

            def inv_body(v0, carry):
                for u in range(8):
                    v = v0 * 8 + u
                    bm16 = bm_v[pl.ds(v * _LANES, _LANES)]
                    m = (bm16 >= base) & (bm16 < base + _BW)
                    r = bm16 - base
                    plsc.store_scatter(slot2d, [r >> 7, r & 127],
                                       v * _LANES + lane, mask=m)
                return carry

            lax.fori_loop(0, _NSLOT // _LANES // 8, inv_body, jnp.int32(0))

        invert(bmap_u, slot_u)
        invert(bmap_i, slot_i)

        cps = []
        for j in range(4):
            sl = pl.ds(j * 128, 128)
            cps.append(pltpu.async_copy(
                ug.at[slot_u.at[j]], urows.at[sl], sem))
            cps.append(pltpu.async_copy(
                ig.at[slot_i.at[j]], irows.at[sl], sem))
        for c in cps:
            c.wait()

        def group(g, carry):
            row = g * _LANES + lane
            sl16 = pl.ds(g * _LANES, _LANES)
            acc = ubf[sl16] + ibf[sl16]
            for d in range(_D):
                col = jnp.full((_LANES,), d, jnp.int32)
                acc = acc + (plsc.load_gather(urows, [row, col])
                             * plsc.load_gather(irows, [row, col]))
            out_v[sl16] = acc
            idu = uidf[sl16]
            idi = iidf[sl16]
            mu = idu >= _TAIL
            mi = idi >= _TAIL
            anyt = jnp.sum((mu | mi).astype(jnp.int32))

            @pl.when(anyt > 0)
            def _():
                tru = idu - _TAIL
                tri = idi - _TAIL
                acc2 = ubf[sl16] + ibf[sl16]
                for d in range(_D):
                    col = jnp.full((_LANES,), d, jnp.int32)
                    uv = plsc.load_gather(urows, [row, col])
                    utv = plsc.load_gather(ut_v, [tru, col], mask=mu)
                    uv = jnp.where(mu, utv, uv)
                    iv = plsc.load_gather(irows, [row, col])
                    itv = plsc.load_gather(it_v, [tri, col], mask=mi)
                    iv = jnp.where(mi, itv, iv)
                    acc2 = acc2 + uv * iv
                out_v[sl16] = jnp.where(mu | mi, acc2, out_v[sl16])

            return carry

        lax.fori_loop(0, _BW // _LANES, group, jnp.int32(0))
        pltpu.sync_copy(out_v, out_hbm.at[pl.ds(base, _BW)])

    return k2


def kernel(user_ids, item_ids, user_emb, item_emb, user_bias, item_bias):
    ug1, ig1, bmap_u, bmap_i = _build_k1()(
        user_ids, item_ids, user_emb.T, item_emb.T)
    tail_idx = jnp.arange(_TAIL, _V, dtype=jnp.int32)
    u_tail = jnp.take(user_emb, tail_idx, axis=0)
    i_tail = jnp.take(item_emb, tail_idx, axis=0)
    return _build_k2()(
        user_ids, item_ids,
        ug1.reshape(_NSLOT, _D), ig1.reshape(_NSLOT, _D),
        bmap_u, bmap_i, u_tail, i_tail,
        user_bias.T, item_bias.T)
